# baseline (speedup 1.0000x reference)
import functools

import jax
import jax.numpy as jnp
from jax import lax
from jax.experimental import pallas as pl
from jax.experimental.pallas import tpu as pltpu
from jax.experimental.pallas import tpu_sc as plsc

N_FACTORS = 32
BATCH = 16384
NUM_CORES = 2
NUM_SUBCORES = 16
NUM_WORKERS = NUM_CORES * NUM_SUBCORES
LANES = 16
B_PER_W = BATCH // NUM_WORKERS
CHUNK = 128
CHUNKS = B_PER_W // CHUNK
GROUPS = CHUNK // LANES
ROWS_PER_GROUP = 128 // N_FACTORS

_mesh = plsc.VectorSubcoreMesh(core_axis_name="c", subcore_axis_name="s")


@functools.partial(
    pl.kernel,
    mesh=_mesh,
    out_type=jax.ShapeDtypeStruct((BATCH,), jnp.float32),
    compiler_params=pltpu.CompilerParams(needs_layout_passes=False),
    scratch_types=[
        pltpu.VMEM((CHUNKS, CHUNK), jnp.int32),
        pltpu.VMEM((CHUNKS, CHUNK), jnp.int32),
        pltpu.VMEM((CHUNKS, CHUNK), jnp.int32),
        pltpu.VMEM((CHUNKS, CHUNK), jnp.int32),
        pltpu.VMEM((2, CHUNK, 128), jnp.float32),
        pltpu.VMEM((2, CHUNK, 128), jnp.float32),
        pltpu.VMEM((B_PER_W,), jnp.float32),
        pltpu.SemaphoreType.DMA,
        pltpu.SemaphoreType.DMA,
    ],
)
def _mf_sc(user_hbm, item_hbm, uf_hbm, vf_hbm, out_hbm,
           uidx, iidx, udiv, idiv, urows, vrows, outv, sem0, sem1):
    wid = lax.axis_index("s") * NUM_CORES + lax.axis_index("c")

    pltpu.sync_copy(user_hbm.at[pl.ds(wid * CHUNKS, CHUNKS)], uidx)
    pltpu.sync_copy(item_hbm.at[pl.ds(wid * CHUNKS, CHUNKS)], iidx)

    for c in range(CHUNKS):
        for j in range(CHUNK // LANES):
            s = pl.ds(j * LANES, LANES)
            udiv[c, s] = jnp.right_shift(uidx[c, s], 2)
            idiv[c, s] = jnp.right_shift(iidx[c, s], 2)

    sems = (sem0, sem1)

    def fire(c):
        slot = c % 2
        return (
            pltpu.async_copy(uf_hbm.at[udiv.at[c]], urows.at[slot], sems[slot]),
            pltpu.async_copy(vf_hbm.at[idiv.at[c]], vrows.at[slot], sems[slot]),
        )

    lanes = lax.iota(jnp.int32, LANES)
    inflight = {0: fire(0), 1: fire(1)}

    for c in range(CHUNKS):
        slot = c % 2
        cu, cv = inflight.pop(c)
        cu.wait()
        cv.wait()
        ubuf = urows.at[slot]
        vbuf = vrows.at[slot]
        for g in range(GROUPS):
            s = pl.ds(g * LANES, LANES)
            ucol = jnp.bitwise_and(uidx[c, s], ROWS_PER_GROUP - 1) * N_FACTORS
            icol = jnp.bitwise_and(iidx[c, s], ROWS_PER_GROUP - 1) * N_FACTORS
            rows16 = g * LANES + lanes
            acc = jnp.zeros((LANES,), jnp.float32)
            for f in range(N_FACTORS):
                uu = plsc.load_gather(ubuf, [rows16, ucol + f])
                vv = plsc.load_gather(vbuf, [rows16, icol + f])
                acc = acc + uu * vv
            outv[pl.ds(c * CHUNK + g * LANES, LANES)] = acc
        if c + 2 < CHUNKS:
            inflight[c + 2] = fire(c + 2)

    pltpu.sync_copy(outv, out_hbm.at[pl.ds(wid * B_PER_W, B_PER_W)])


def kernel(user, item, user_factors, item_factors):
    u2 = user.reshape(NUM_WORKERS * CHUNKS, CHUNK)
    i2 = item.reshape(NUM_WORKERS * CHUNKS, CHUNK)
    uf2 = user_factors.reshape(-1, 128)
    vf2 = item_factors.reshape(-1, 128)
    return _mf_sc(u2, i2, uf2, vf2)


# speedup vs baseline: 3.5752x; 3.5752x over previous
import functools

import jax
import jax.numpy as jnp
from jax import lax
from jax.experimental import pallas as pl
from jax.experimental.pallas import tpu as pltpu
from jax.experimental.pallas import tpu_sc as plsc

N_FACTORS = 32
BATCH = 16384
NUM_CORES = 2
NUM_SUBCORES = 16
NUM_WORKERS = NUM_CORES * NUM_SUBCORES
LANES = 16
B_PER_W = BATCH // NUM_WORKERS
IDX_ROWS = 4
IDX_COLS = B_PER_W // IDX_ROWS
GROUPS = B_PER_W // LANES

_mesh = plsc.VectorSubcoreMesh(core_axis_name="c", subcore_axis_name="s")


@functools.partial(
    pl.kernel,
    mesh=_mesh,
    out_type=jax.ShapeDtypeStruct((BATCH,), jnp.float32),
    compiler_params=pltpu.CompilerParams(needs_layout_passes=False),
    scratch_types=[
        pltpu.VMEM((IDX_ROWS, IDX_COLS), jnp.int32),
        pltpu.VMEM((IDX_ROWS, IDX_COLS), jnp.int32),
        pltpu.VMEM((LANES, N_FACTORS, 128), jnp.float32),
        pltpu.VMEM((LANES, N_FACTORS), jnp.float32),
        pltpu.VMEM((B_PER_W,), jnp.float32),
        pltpu.SemaphoreType.DMA,
    ],
)
def _mf_sc(user_hbm, item_hbm, uft_hbm, vft_hbm, out_hbm,
           uidx, iidx, blk, ucols, outv, sem):
    wid = lax.axis_index("s") * NUM_CORES + lax.axis_index("c")

    pltpu.sync_copy(user_hbm.at[pl.ds(wid * IDX_ROWS, IDX_ROWS)], uidx)
    pltpu.sync_copy(item_hbm.at[pl.ds(wid * IDX_ROWS, IDX_ROWS)], iidx)

    lanes = lax.iota(jnp.int32, LANES)

    def body(g, carry):
        r = jnp.right_shift(g, 3)
        c = pl.multiple_of(jnp.bitwise_and(g, 7) * LANES, LANES)
        uvec = uidx[r, pl.ds(c, LANES)]
        vvec = iidx[r, pl.ds(c, LANES)]

        for k in range(LANES):
            ub = pl.multiple_of(jnp.bitwise_and(uvec[k], -128), 128)
            pltpu.async_copy(uft_hbm.at[:, pl.ds(ub, 128)], blk.at[k], sem)
        for k in range(LANES):
            pltpu.make_async_copy(uft_hbm.at[:, pl.ds(0, 128)], blk.at[k],
                                  sem).wait()
        for k in range(LANES):
            ucol = jnp.broadcast_to(jnp.bitwise_and(uvec[k], 127), (LANES,))
            u0 = plsc.load_gather(blk.at[k], [lanes, ucol])
            u1 = plsc.load_gather(blk.at[k], [lanes + LANES, ucol])
            ucols[k, pl.ds(0, LANES)] = u0
            ucols[k, pl.ds(LANES, LANES)] = u1

        for k in range(LANES):
            vb = pl.multiple_of(jnp.bitwise_and(vvec[k], -128), 128)
            pltpu.async_copy(vft_hbm.at[:, pl.ds(vb, 128)], blk.at[k], sem)
        for k in range(LANES):
            pltpu.make_async_copy(vft_hbm.at[:, pl.ds(0, 128)], blk.at[k],
                                  sem).wait()
        acc = jnp.zeros((LANES,), jnp.float32)
        for k in range(LANES):
            vcol = jnp.broadcast_to(jnp.bitwise_and(vvec[k], 127), (LANES,))
            v0 = plsc.load_gather(blk.at[k], [lanes, vcol])
            v1 = plsc.load_gather(blk.at[k], [lanes + LANES, vcol])
            p = ucols[k, pl.ds(0, LANES)] * v0 + ucols[k, pl.ds(LANES, LANES)] * v1
            s = jnp.sum(p)
            acc = jnp.where(lanes == k, s, acc)

        base = pl.multiple_of(g * LANES, LANES)
        outv[pl.ds(base, LANES)] = acc
        return carry

    lax.fori_loop(0, GROUPS, body, 0)

    pltpu.sync_copy(outv, out_hbm.at[pl.ds(wid * B_PER_W, B_PER_W)])


def kernel(user, item, user_factors, item_factors):
    u2 = user.reshape(NUM_WORKERS * IDX_ROWS, IDX_COLS)
    i2 = item.reshape(NUM_WORKERS * IDX_ROWS, IDX_COLS)
    return _mf_sc(u2, i2, user_factors.T, item_factors.T)
